# initial kernel scaffold (unmeasured)
import jax
import jax.numpy as jnp
from jax import lax
from jax.experimental import pallas as pl
from jax.experimental.pallas import tpu as pltpu


def kernel(
    x,
):
    def body(*refs):
        pass

    out_shape = jax.ShapeDtypeStruct(..., jnp.float32)
    return pl.pallas_call(body, out_shape=out_shape)(...)


# baseline (device time: 3362940 ns/iter reference)
import jax
import jax.numpy as jnp
from jax import lax
from jax.experimental import pallas as pl
from jax.experimental.pallas import tpu as pltpu

P = 8
LOGP = 3
TN = 256
C = 512


def _cmp_pass(work, j, k, my_i, m):
    nchunk = m // C
    if j < C:
        g = C // (2 * j)

        def chunk(c, carry):
            x = work[pl.ds(c * C, C), :]
            x = x.reshape(g, 2, j, TN)
            a = x[:, 0]
            b = x[:, 1]
            lo = jnp.minimum(a, b)
            hi = jnp.maximum(a, b)
            if k <= m // 2:
                base = (
                    lax.broadcasted_iota(jnp.int32, (g, 1, 1), 0) * (2 * j)
                    + c * C
                )
                up = (base & k) == 0
            else:
                up = ((my_i * m) & k) == 0
            n0 = jnp.where(up, lo, hi)
            n1 = jnp.where(up, hi, lo)
            work[pl.ds(c * C, C), :] = jnp.stack([n0, n1], axis=1).reshape(
                C, TN
            )
            return carry

        lax.fori_loop(0, nchunk, chunk, 0)
    else:
        job = j // C

        def pair(p_, carry):
            g_idx = p_ // job
            within = p_ % job
            row_a = g_idx * 2 * j + within * C
            row_b = row_a + j
            a = work[pl.ds(row_a, C), :]
            b = work[pl.ds(row_b, C), :]
            lo = jnp.minimum(a, b)
            hi = jnp.maximum(a, b)
            if k <= m // 2:
                up = (row_a & k) == 0
            else:
                up = ((my_i * m) & k) == 0
            work[pl.ds(row_a, C), :] = jnp.where(up, lo, hi)
            work[pl.ds(row_b, C), :] = jnp.where(up, hi, lo)
            return carry

        lax.fori_loop(0, m // (2 * C), pair, 0)


def kernel(x):
    m, n = x.shape
    logm = m.bit_length() - 1
    nt = n // TN

    def body(x_hbm, o_hbm, work, comm, copy_sem, send_sems, recv_sems,
             ready_sems):
        my_i = lax.axis_index("i")

        def tile_body(t, carry):
            col = t * TN
            cp_in = pltpu.make_async_copy(
                x_hbm.at[:, pl.ds(col, TN)], work, copy_sem
            )
            cp_in.start()
            cp_in.wait()

            for lk in range(1, logm + 1):
                k = 1 << lk
                for lj in range(lk - 1, -1, -1):
                    _cmp_pass(work, 1 << lj, k, my_i, m)

            e = 0
            for lkd in range(1, LOGP + 1):
                k = m << lkd
                up = ((my_i * m) & k) == 0
                for l_j in range(lkd - 1, -1, -1):
                    jdev = 1 << l_j
                    partner = my_i ^ jdev
                    keep_min = (((my_i & jdev) == 0) == up)
                    slot = e % 2
                    hs = (t * 6 + e) % 8

                    pl.semaphore_signal(
                        ready_sems.at[hs],
                        inc=1,
                        device_id=(partner,),
                        device_id_type=pl.DeviceIdType.MESH,
                    )
                    pl.semaphore_wait(ready_sems.at[hs], 1)

                    rdma = pltpu.make_async_remote_copy(
                        src_ref=work,
                        dst_ref=comm.at[slot],
                        send_sem=send_sems.at[slot],
                        recv_sem=recv_sems.at[slot],
                        device_id=(partner,),
                        device_id_type=pl.DeviceIdType.MESH,
                    )
                    rdma.start()
                    rdma.wait()

                    def xchunk(c, carry2, _slot=slot, _keep=keep_min):
                        a = work[pl.ds(c * C, C), :]
                        b = comm[_slot, pl.ds(c * C, C), :]
                        work[pl.ds(c * C, C), :] = jnp.where(
                            _keep, jnp.minimum(a, b), jnp.maximum(a, b)
                        )
                        return carry2

                    lax.fori_loop(0, m // C, xchunk, 0)
                    e += 1

                for lj in range(logm - 1, -1, -1):
                    _cmp_pass(work, 1 << lj, k, my_i, m)

            cp_out = pltpu.make_async_copy(
                work, o_hbm.at[:, pl.ds(col, TN)], copy_sem
            )
            cp_out.start()
            cp_out.wait()
            return carry

        lax.fori_loop(0, nt, tile_body, 0)

    return pl.pallas_call(
        body,
        out_shape=jax.ShapeDtypeStruct((m, n), x.dtype),
        in_specs=[pl.BlockSpec(memory_space=pl.ANY)],
        out_specs=pl.BlockSpec(memory_space=pl.ANY),
        scratch_shapes=[
            pltpu.VMEM((m, TN), jnp.float32),
            pltpu.VMEM((2, m, TN), jnp.float32),
            pltpu.SemaphoreType.DMA,
            pltpu.SemaphoreType.DMA((2,)),
            pltpu.SemaphoreType.DMA((2,)),
            pltpu.SemaphoreType.REGULAR((8,)),
        ],
    )(x)
